# baseline (speedup 1.0000x reference)
import functools

import jax
import jax.numpy as jnp
from jax import lax
from jax.experimental import pallas as pl
from jax.experimental.pallas import tpu as pltpu
from jax.experimental.pallas import tpu_sc as plsc

N = 10000
E = 320000
D = 128
H = 128
C = 2
B = 64

AW = 144
NC = 2
NS = 16
NW = NC * NS
EPW = E // NW
CHUNK = 40
NCHUNK = EPW // CHUNK
NP = 10240
RPT = NP // NS

RB = 1000
NRB = N // RB
CP = 8


def _sc_aggregate(xa, src_r, dst_r, zeros_hbm):
    mesh = plsc.VectorSubcoreMesh(core_axis_name="c", subcore_axis_name="s")

    @functools.partial(
        pl.kernel,
        mesh=mesh,
        compiler_params=pltpu.CompilerParams(use_tc_tiling_on_sc=False),
        out_type=jax.ShapeDtypeStruct((NC, NP, AW), jnp.float32),
        scratch_types=[
            pltpu.VMEM((NCHUNK, CHUNK), jnp.int32),
            pltpu.VMEM((NCHUNK, CHUNK), jnp.int32),
            pltpu.VMEM((CHUNK, AW), jnp.float32),
            pltpu.VMEM((CHUNK, AW), jnp.float32),
            pltpu.VMEM_SHARED((NP, AW), jnp.float32),
            pltpu.SemaphoreType.DMA,
            pltpu.SemaphoreType.DMA,
            pltpu.SemaphoreType.DMA,
            pltpu.SemaphoreType.DMA,
        ],
    )
    def k(xa_hbm, src_hbm, dst_hbm, z_hbm, out_hbm, src_v, dst_v,
          buf0, buf1, acc_sh, sem0, sem1, ssem0, ssem1):
        c = lax.axis_index("c")
        s = lax.axis_index("s")
        wid = c * NS + s

        pltpu.sync_copy(z_hbm.at[pl.ds(s * RPT, RPT)],
                        acc_sh.at[pl.ds(s * RPT, RPT)])
        pltpu.sync_copy(src_hbm.at[wid], src_v)
        pltpu.sync_copy(dst_hbm.at[wid], dst_v)
        plsc.subcore_barrier()

        def issue(j, buf_, sem_):
            pltpu.async_copy(xa_hbm.at[src_v.at[j]], buf_, sem_)

        def wait_g(buf_, sem_):
            pltpu.make_async_copy(xa_hbm.at[src_v.at[0]], buf_, sem_).wait()

        def scat_async(j, buf_, sem_):
            pltpu.async_copy(buf_, acc_sh.at[dst_v.at[j]], sem_, add=True)

        def wait_s(buf_, sem_):
            pltpu.make_async_copy(buf_, acc_sh.at[dst_v.at[0]], sem_).wait()

        issue(0, buf0, sem0)
        issue(1, buf1, sem1)

        def body(i, carry):
            j = 2 * i
            wait_g(buf0, sem0)
            scat_async(j, buf0, ssem0)
            wait_g(buf1, sem1)
            scat_async(j + 1, buf1, ssem1)
            wait_s(buf0, ssem0)
            issue(j + 2, buf0, sem0)
            wait_s(buf1, ssem1)
            issue(j + 3, buf1, sem1)
            return carry

        lax.fori_loop(0, (NCHUNK - 2) // 2, body, 0)
        wait_g(buf0, sem0)
        pltpu.sync_copy(buf0, acc_sh.at[dst_v.at[NCHUNK - 2]], add=True)
        wait_g(buf1, sem1)
        pltpu.sync_copy(buf1, acc_sh.at[dst_v.at[NCHUNK - 1]], add=True)
        plsc.subcore_barrier()

        pltpu.sync_copy(acc_sh.at[pl.ds(s * RPT, RPT)],
                        out_hbm.at[c, pl.ds(s * RPT, RPT)])

    return k(xa, src_r, dst_r, zeros_hbm)


def _tc_body(bounds_ref, part_ref, x_ref, batch_ref, wlt_ref, wrt_ref,
             bl_ref, w2t_ref, b2_ref, out_ref, acc_ref):
    i = pl.program_id(0)

    @pl.when(i == 0)
    def _():
        acc_ref[...] = jnp.zeros_like(acc_ref)

    p = part_ref[0] + part_ref[1]
    ssum = p[:, :D]
    cnt = p[:, D:D + 1]
    mean = ssum / jnp.maximum(cnt, 1.0)
    z = jnp.dot(mean, wlt_ref[...], preferred_element_type=jnp.float32)
    z = z + jnp.dot(x_ref[...], wrt_ref[...], preferred_element_type=jnp.float32)
    z = z + bl_ref[...]

    bcol = batch_ref[0]
    gmin = bounds_ref[0, i]
    gmax = bounds_ref[1, i]
    for off in range(B):
        g = gmin + off

        @pl.when(g <= gmax)
        def _(g=g):
            zm = jnp.where(bcol == g, z, 0.0)
            contrib = jnp.max(zm, axis=0, keepdims=True)
            cur = acc_ref[pl.ds(g, 1), :]
            acc_ref[pl.ds(g, 1), :] = jnp.maximum(cur, contrib)

    @pl.when(i == NRB - 1)
    def _():
        pooled = acc_ref[...]
        logits = jnp.dot(pooled, w2t_ref[...],
                         preferred_element_type=jnp.float32) + b2_ref[...]
        col = lax.broadcasted_iota(jnp.int32, (B, CP), 1)
        logits = jnp.where(col < C, logits, -jnp.inf)
        mx = jnp.max(logits, axis=-1, keepdims=True)
        sh = logits - mx
        lse = jnp.log(jnp.sum(jnp.exp(sh), axis=-1, keepdims=True))
        out_ref[...] = (sh - lse)[:, :C]


def _tc_head(bounds, partials, x, batch3, wlt, wrt, bl, w2t, b2p):
    grid_spec = pltpu.PrefetchScalarGridSpec(
        num_scalar_prefetch=1,
        grid=(NRB,),
        in_specs=[
            pl.BlockSpec((NC, RB, AW), lambda i, b_: (0, i, 0)),
            pl.BlockSpec((RB, D), lambda i, b_: (i, 0)),
            pl.BlockSpec((1, RB, 1), lambda i, b_: (i, 0, 0)),
            pl.BlockSpec((D, H), lambda i, b_: (0, 0)),
            pl.BlockSpec((D, H), lambda i, b_: (0, 0)),
            pl.BlockSpec((1, H), lambda i, b_: (0, 0)),
            pl.BlockSpec((H, CP), lambda i, b_: (0, 0)),
            pl.BlockSpec((1, CP), lambda i, b_: (0, 0)),
        ],
        out_specs=pl.BlockSpec((B, C), lambda i, b_: (0, 0)),
        scratch_shapes=[pltpu.VMEM((B, H), jnp.float32)],
    )
    return pl.pallas_call(
        _tc_body,
        grid_spec=grid_spec,
        out_shape=jax.ShapeDtypeStruct((B, C), jnp.float32),
    )(bounds, partials, x, batch3, wlt, wrt, bl, w2t, b2p)


def kernel(x, edge_index, batch, W_l, b_l, W_r, W2, b2):
    xa = jnp.concatenate(
        [x, jnp.ones((N, 1), jnp.float32), jnp.zeros((N, AW - D - 1), jnp.float32)],
        axis=1)
    src_r = edge_index[0].reshape(NW, NCHUNK, CHUNK)
    dst_r = edge_index[1].reshape(NW, NCHUNK, CHUNK)
    zeros_hbm = jnp.zeros((NP, AW), jnp.float32)

    partials = _sc_aggregate(xa, src_r, dst_r, zeros_hbm)

    batch2 = batch.reshape(NRB, RB)
    bounds = jnp.stack([batch2[:, 0], batch2[:, -1]])
    batch3 = batch.reshape(NRB, RB, 1)
    wlt = W_l.T
    wrt = W_r.T
    bl = b_l.reshape(1, H)
    w2t = jnp.zeros((H, CP), jnp.float32).at[:, :C].set(W2.T)
    b2p = jnp.zeros((1, CP), jnp.float32).at[0, :C].set(b2)

    return _tc_head(bounds, partials, x, batch3, wlt, wrt, bl, w2t, b2p)


# speedup vs baseline: 1.3352x; 1.3352x over previous
import functools

import jax
import jax.numpy as jnp
from jax import lax
from jax.experimental import pallas as pl
from jax.experimental.pallas import tpu as pltpu
from jax.experimental.pallas import tpu_sc as plsc

N = 10000
E = 320000
D = 128
H = 128
C = 2
B = 64

AW = 144
NC = 2
NS = 16
NW = NC * NS
EPW = E // NW
CHUNK = 40
NCHUNK = EPW // CHUNK
NP = 10240
RPT = NP // NS

RB = 1000
NRB = N // RB
CP = 8


def _sc_aggregate(xa, src_r, dst_r, zeros_hbm):
    mesh = plsc.VectorSubcoreMesh(core_axis_name="c", subcore_axis_name="s")

    @functools.partial(
        pl.kernel,
        mesh=mesh,
        compiler_params=pltpu.CompilerParams(use_tc_tiling_on_sc=False),
        out_type=jax.ShapeDtypeStruct((NC, NP, AW), jnp.float32),
        scratch_types=[
            pltpu.VMEM((NCHUNK, CHUNK), jnp.int32),
            pltpu.VMEM((NCHUNK, CHUNK), jnp.int32),
            pltpu.VMEM((CHUNK, AW), jnp.float32),
            pltpu.VMEM((CHUNK, AW), jnp.float32),
            pltpu.VMEM((CHUNK, AW), jnp.float32),
            pltpu.VMEM_SHARED((NP, AW), jnp.float32),
            pltpu.SemaphoreType.DMA,
            pltpu.SemaphoreType.DMA,
            pltpu.SemaphoreType.DMA,
        ],
    )
    def k(xa_hbm, src_hbm, dst_hbm, z_hbm, out_hbm, src_v, dst_v,
          buf0, buf1, buf2, acc_sh, sem0, sem1, sem2):
        c = lax.axis_index("c")
        s = lax.axis_index("s")
        wid = c * NS + s

        pltpu.sync_copy(z_hbm.at[pl.ds(s * RPT, RPT)],
                        acc_sh.at[pl.ds(s * RPT, RPT)])
        pltpu.sync_copy(src_hbm.at[wid], src_v)
        pltpu.sync_copy(dst_hbm.at[wid], dst_v)
        plsc.subcore_barrier()

        def issue(j, buf_, sem_):
            pltpu.async_copy(xa_hbm.at[src_v.at[j]], buf_, sem_)

        def wait_g(buf_, sem_):
            pltpu.make_async_copy(xa_hbm.at[src_v.at[0]], buf_, sem_).wait()

        def scat(j, buf_):
            pltpu.sync_copy(buf_, acc_sh.at[dst_v.at[j]], add=True)

        issue(0, buf0, sem0)
        issue(1, buf1, sem1)
        issue(2, buf2, sem2)

        def body(i, carry):
            j = 3 * i
            wait_g(buf0, sem0)
            scat(j, buf0)
            issue(j + 3, buf0, sem0)
            wait_g(buf1, sem1)
            scat(j + 1, buf1)
            issue(j + 4, buf1, sem1)
            wait_g(buf2, sem2)
            scat(j + 2, buf2)
            issue(j + 5, buf2, sem2)
            return carry

        lax.fori_loop(0, (NCHUNK - 4) // 3, body, 0)
        wait_g(buf0, sem0)
        scat(NCHUNK - 4, buf0)
        issue(NCHUNK - 1, buf0, sem0)
        wait_g(buf1, sem1)
        scat(NCHUNK - 3, buf1)
        wait_g(buf2, sem2)
        scat(NCHUNK - 2, buf2)
        wait_g(buf0, sem0)
        scat(NCHUNK - 1, buf0)
        plsc.subcore_barrier()

        pltpu.sync_copy(acc_sh.at[pl.ds(s * RPT, RPT)],
                        out_hbm.at[c, pl.ds(s * RPT, RPT)])

    return k(xa, src_r, dst_r, zeros_hbm)


def _tc_body(bounds_ref, part_ref, x_ref, batch_ref, wlt_ref, wrt_ref,
             bl_ref, w2t_ref, b2_ref, out_ref, acc_ref):
    i = pl.program_id(0)

    @pl.when(i == 0)
    def _():
        acc_ref[...] = jnp.zeros_like(acc_ref)

    p = part_ref[0] + part_ref[1]
    ssum = p[:, :D]
    cnt = p[:, D:D + 1]
    mean = ssum / jnp.maximum(cnt, 1.0)
    z = jnp.dot(mean, wlt_ref[...], preferred_element_type=jnp.float32)
    z = z + jnp.dot(x_ref[...], wrt_ref[...], preferred_element_type=jnp.float32)
    z = z + bl_ref[...]

    bcol = batch_ref[0]
    gmin = bounds_ref[0, i]
    gmax = bounds_ref[1, i]
    for off in range(B):
        g = gmin + off

        @pl.when(g <= gmax)
        def _(g=g):
            zm = jnp.where(bcol == g, z, 0.0)
            contrib = jnp.max(zm, axis=0, keepdims=True)
            cur = acc_ref[pl.ds(g, 1), :]
            acc_ref[pl.ds(g, 1), :] = jnp.maximum(cur, contrib)

    @pl.when(i == NRB - 1)
    def _():
        pooled = acc_ref[...]
        logits = jnp.dot(pooled, w2t_ref[...],
                         preferred_element_type=jnp.float32) + b2_ref[...]
        col = lax.broadcasted_iota(jnp.int32, (B, CP), 1)
        logits = jnp.where(col < C, logits, -jnp.inf)
        mx = jnp.max(logits, axis=-1, keepdims=True)
        sh = logits - mx
        lse = jnp.log(jnp.sum(jnp.exp(sh), axis=-1, keepdims=True))
        out_ref[...] = (sh - lse)[:, :C]


def _tc_head(bounds, partials, x, batch3, wlt, wrt, bl, w2t, b2p):
    grid_spec = pltpu.PrefetchScalarGridSpec(
        num_scalar_prefetch=1,
        grid=(NRB,),
        in_specs=[
            pl.BlockSpec((NC, RB, AW), lambda i, b_: (0, i, 0)),
            pl.BlockSpec((RB, D), lambda i, b_: (i, 0)),
            pl.BlockSpec((1, RB, 1), lambda i, b_: (i, 0, 0)),
            pl.BlockSpec((D, H), lambda i, b_: (0, 0)),
            pl.BlockSpec((D, H), lambda i, b_: (0, 0)),
            pl.BlockSpec((1, H), lambda i, b_: (0, 0)),
            pl.BlockSpec((H, CP), lambda i, b_: (0, 0)),
            pl.BlockSpec((1, CP), lambda i, b_: (0, 0)),
        ],
        out_specs=pl.BlockSpec((B, C), lambda i, b_: (0, 0)),
        scratch_shapes=[pltpu.VMEM((B, H), jnp.float32)],
    )
    return pl.pallas_call(
        _tc_body,
        grid_spec=grid_spec,
        out_shape=jax.ShapeDtypeStruct((B, C), jnp.float32),
    )(bounds, partials, x, batch3, wlt, wrt, bl, w2t, b2p)


def kernel(x, edge_index, batch, W_l, b_l, W_r, W2, b2):
    xa = jnp.concatenate(
        [x, jnp.ones((N, 1), jnp.float32), jnp.zeros((N, AW - D - 1), jnp.float32)],
        axis=1)
    src_r = edge_index[0].reshape(NW, NCHUNK, CHUNK)
    dst_r = edge_index[1].reshape(NW, NCHUNK, CHUNK)
    zeros_hbm = jnp.zeros((NP, AW), jnp.float32)

    partials = _sc_aggregate(xa, src_r, dst_r, zeros_hbm)

    batch2 = batch.reshape(NRB, RB)
    bounds = jnp.stack([batch2[:, 0], batch2[:, -1]])
    batch3 = batch.reshape(NRB, RB, 1)
    wlt = W_l.T
    wrt = W_r.T
    bl = b_l.reshape(1, H)
    w2t = jnp.zeros((H, CP), jnp.float32).at[:, :C].set(W2.T)
    b2p = jnp.zeros((1, CP), jnp.float32).at[0, :C].set(b2)

    return _tc_head(bounds, partials, x, batch3, wlt, wrt, bl, w2t, b2p)


# speedup vs baseline: 1.5969x; 1.1959x over previous
import functools

import jax
import jax.numpy as jnp
from jax import lax
from jax.experimental import pallas as pl
from jax.experimental.pallas import tpu as pltpu
from jax.experimental.pallas import tpu_sc as plsc

N = 10000
E = 320000
D = 128
H = 128
C = 2
B = 64

AW = 144
NC = 2
NS = 16
NW = NC * NS
EPW = E // NW
CHUNK = 40
NCHUNK = EPW // CHUNK
NP = 10240
RPT = NP // NS

RB = 1000
NRB = N // RB
CP = 8
CW = 16


def _sc_aggregate(x, src_r, dst_r):
    mesh = plsc.VectorSubcoreMesh(core_axis_name="c", subcore_axis_name="s")

    @functools.partial(
        pl.kernel,
        mesh=mesh,
        compiler_params=pltpu.CompilerParams(use_tc_tiling_on_sc=False),
        out_type=[jax.ShapeDtypeStruct((NC, NP, D), jnp.float32),
                  jax.ShapeDtypeStruct((NC, NP, CW), jnp.float32)],
        scratch_types=[
            pltpu.VMEM((NCHUNK, CHUNK), jnp.int32),
            pltpu.VMEM((NCHUNK, CHUNK), jnp.int32),
            pltpu.VMEM((CHUNK, D), jnp.float32),
            pltpu.VMEM((CHUNK, D), jnp.float32),
            pltpu.VMEM((CHUNK, D), jnp.float32),
            pltpu.VMEM((CHUNK, CW), jnp.float32),
            pltpu.VMEM_SHARED((NP, D), jnp.float32),
            pltpu.VMEM_SHARED((NP, CW), jnp.float32),
            pltpu.SemaphoreType.DMA,
            pltpu.SemaphoreType.DMA,
            pltpu.SemaphoreType.DMA,
            pltpu.SemaphoreType.DMA,
        ],
    )
    def k(x_hbm, src_hbm, dst_hbm, out_hbm, cnt_hbm, src_v, dst_v,
          buf0, buf1, buf2, obuf, acc_sh, cnt_sh, sem0, sem1, sem2, csem):
        c = lax.axis_index("c")
        s = lax.axis_index("s")
        wid = c * NS + s

        zero16 = jnp.zeros((16,), jnp.float32)

        def zb(kk, carry):
            buf0[kk // 8, pl.ds((kk % 8) * 16, 16)] = zero16
            return carry

        lax.fori_loop(0, CHUNK * (D // 16), zb, 0)

        def zo(r, carry):
            obuf[r, pl.ds(0, 16)] = zero16
            return carry

        lax.fori_loop(0, CHUNK, zo, 0)

        def zacc(kk, carry):
            pltpu.sync_copy(buf0, acc_sh.at[pl.ds(s * RPT + kk * CHUNK, CHUNK)])
            pltpu.sync_copy(obuf, cnt_sh.at[pl.ds(s * RPT + kk * CHUNK, CHUNK)])
            return carry

        lax.fori_loop(0, RPT // CHUNK, zacc, 0)

        pltpu.sync_copy(src_hbm.at[wid], src_v)
        pltpu.sync_copy(dst_hbm.at[wid], dst_v)

        one16 = jnp.where(lax.iota(jnp.int32, 16) == 0, 1.0, 0.0)

        def seto(r, carry):
            obuf[r, pl.ds(0, 16)] = one16
            return carry

        lax.fori_loop(0, CHUNK, seto, 0)
        plsc.subcore_barrier()

        def issue(j, buf_, sem_):
            pltpu.async_copy(x_hbm.at[src_v.at[j]], buf_, sem_)

        def wait_g(buf_, sem_):
            pltpu.make_async_copy(x_hbm.at[src_v.at[0]], buf_, sem_).wait()

        def scat(j, buf_):
            pltpu.sync_copy(buf_, acc_sh.at[dst_v.at[j]], add=True)

        def cnt_issue(j):
            pltpu.async_copy(obuf, cnt_sh.at[dst_v.at[j]], csem, add=True)

        def cnt_wait():
            pltpu.make_async_copy(obuf, cnt_sh.at[dst_v.at[0]], csem).wait()

        issue(0, buf0, sem0)
        issue(1, buf1, sem1)
        issue(2, buf2, sem2)

        def body(i, carry):
            j = 3 * i

            @pl.when(i > 0)
            def _():
                cnt_wait()
                cnt_wait()
                cnt_wait()

            wait_g(buf0, sem0)
            scat(j, buf0)
            issue(j + 3, buf0, sem0)
            cnt_issue(j)
            wait_g(buf1, sem1)
            scat(j + 1, buf1)
            issue(j + 4, buf1, sem1)
            cnt_issue(j + 1)
            wait_g(buf2, sem2)
            scat(j + 2, buf2)
            issue(j + 5, buf2, sem2)
            cnt_issue(j + 2)
            return carry

        lax.fori_loop(0, (NCHUNK - 4) // 3, body, 0)
        cnt_wait()
        cnt_wait()
        cnt_wait()
        wait_g(buf0, sem0)
        scat(NCHUNK - 4, buf0)
        issue(NCHUNK - 1, buf0, sem0)
        cnt_issue(NCHUNK - 4)
        wait_g(buf1, sem1)
        scat(NCHUNK - 3, buf1)
        cnt_issue(NCHUNK - 3)
        wait_g(buf2, sem2)
        scat(NCHUNK - 2, buf2)
        cnt_issue(NCHUNK - 2)
        wait_g(buf0, sem0)
        scat(NCHUNK - 1, buf0)
        cnt_issue(NCHUNK - 1)
        cnt_wait()
        cnt_wait()
        cnt_wait()
        cnt_wait()
        plsc.subcore_barrier()

        pltpu.sync_copy(acc_sh.at[pl.ds(s * RPT, RPT)],
                        out_hbm.at[c, pl.ds(s * RPT, RPT)])
        pltpu.sync_copy(cnt_sh.at[pl.ds(s * RPT, RPT)],
                        cnt_hbm.at[c, pl.ds(s * RPT, RPT)])

    return k(x, src_r, dst_r)


def _tc_body(bounds_ref, pf_ref, pc_ref, x_ref, batch_ref, wlt_ref, wrt_ref,
             bl_ref, w2t_ref, b2_ref, out_ref, acc_ref):
    i = pl.program_id(0)

    @pl.when(i == 0)
    def _():
        acc_ref[...] = jnp.zeros_like(acc_ref)

    ssum = pf_ref[0] + pf_ref[1]
    pc = pc_ref[0] + pc_ref[1]
    cnt = pc[:, :1]
    mean = ssum / jnp.maximum(cnt, 1.0)
    z = jnp.dot(mean, wlt_ref[...], preferred_element_type=jnp.float32)
    z = z + jnp.dot(x_ref[...], wrt_ref[...], preferred_element_type=jnp.float32)
    z = z + bl_ref[...]

    bcol = batch_ref[0]
    gmin = bounds_ref[0, i]
    gmax = bounds_ref[1, i]
    for off in range(B):
        g = gmin + off

        @pl.when(g <= gmax)
        def _(g=g):
            zm = jnp.where(bcol == g, z, 0.0)
            contrib = jnp.max(zm, axis=0, keepdims=True)
            cur = acc_ref[pl.ds(g, 1), :]
            acc_ref[pl.ds(g, 1), :] = jnp.maximum(cur, contrib)

    @pl.when(i == NRB - 1)
    def _():
        pooled = acc_ref[...]
        logits = jnp.dot(pooled, w2t_ref[...],
                         preferred_element_type=jnp.float32) + b2_ref[...]
        col = lax.broadcasted_iota(jnp.int32, (B, CP), 1)
        logits = jnp.where(col < C, logits, -jnp.inf)
        mx = jnp.max(logits, axis=-1, keepdims=True)
        sh = logits - mx
        lse = jnp.log(jnp.sum(jnp.exp(sh), axis=-1, keepdims=True))
        out_ref[...] = (sh - lse)[:, :C]


def _tc_head(bounds, pfeat, pcnt, x, batch3, wlt, wrt, bl, w2t, b2p):
    grid_spec = pltpu.PrefetchScalarGridSpec(
        num_scalar_prefetch=1,
        grid=(NRB,),
        in_specs=[
            pl.BlockSpec((NC, RB, D), lambda i, b_: (0, i, 0)),
            pl.BlockSpec((NC, RB, CW), lambda i, b_: (0, i, 0)),
            pl.BlockSpec((RB, D), lambda i, b_: (i, 0)),
            pl.BlockSpec((1, RB, 1), lambda i, b_: (i, 0, 0)),
            pl.BlockSpec((D, H), lambda i, b_: (0, 0)),
            pl.BlockSpec((D, H), lambda i, b_: (0, 0)),
            pl.BlockSpec((1, H), lambda i, b_: (0, 0)),
            pl.BlockSpec((H, CP), lambda i, b_: (0, 0)),
            pl.BlockSpec((1, CP), lambda i, b_: (0, 0)),
        ],
        out_specs=pl.BlockSpec((B, C), lambda i, b_: (0, 0)),
        scratch_shapes=[pltpu.VMEM((B, H), jnp.float32)],
    )
    return pl.pallas_call(
        _tc_body,
        grid_spec=grid_spec,
        out_shape=jax.ShapeDtypeStruct((B, C), jnp.float32),
    )(bounds, pfeat, pcnt, x, batch3, wlt, wrt, bl, w2t, b2p)


def kernel(x, edge_index, batch, W_l, b_l, W_r, W2, b2):
    src_r = edge_index[0].reshape(NW, NCHUNK, CHUNK)
    dst_r = edge_index[1].reshape(NW, NCHUNK, CHUNK)

    pfeat, pcnt = _sc_aggregate(x, src_r, dst_r)

    batch2 = batch.reshape(NRB, RB)
    bounds = jnp.stack([batch2[:, 0], batch2[:, -1]])
    batch3 = batch.reshape(NRB, RB, 1)
    wlt = W_l.T
    wrt = W_r.T
    bl = b_l.reshape(1, H)
    w2t = jnp.zeros((H, CP), jnp.float32).at[:, :C].set(W2.T)
    b2p = jnp.zeros((1, CP), jnp.float32).at[0, :C].set(b2)

    return _tc_head(bounds, pfeat, pcnt, x, batch3, wlt, wrt, bl, w2t, b2p)
